# Initial kernel scaffold; baseline (speedup 1.0000x reference)
import jax
import jax.numpy as jnp
from jax.experimental import pallas as pl


def kernel(edges, poses, infos, nodes):
    raise NotImplementedError("write your pallas kernel here")


# speedup vs baseline: 2.1350x; 2.1350x over previous
import functools

import jax
import jax.numpy as jnp
from jax import lax
from jax.experimental import pallas as pl
from jax.experimental.pallas import tpu as pltpu
from jax.experimental.pallas import tpu_sc as plsc

_N_NODES = 100000
_N_EDGES = 1600000
_NW = 32
_C = 512
_S = 128
_G = _C // 16
_NCHUNKS = _N_EDGES // _C
_CPW = -(-_NCHUNKS // _NW)

_HALF_PI = 1.5707963267948966

_ATAN_C = (
    0.9999999937572801, -0.3333313797588716, 0.19993694079563193,
    -0.14211102809331508, 0.10667470050577645, -0.07556856634693432,
    0.04327764436731928, -0.01641277490593999, 0.00293264657519318,
)


def _f32(x):
    return jnp.float32(x)


def _recip(x):
    i = plsc.bitcast(x, jnp.int32)
    i = jnp.int32(0x7EF311C3) - i
    r = plsc.bitcast(i, jnp.float32)
    for _ in range(3):
        r = r * (_f32(2.0) - x * r)
    return r


def _rsqrt(a):
    i = plsc.bitcast(a, jnp.int32)
    i = jnp.int32(0x5F3759DF) - (i >> 1)
    y = plsc.bitcast(i, jnp.float32)
    for _ in range(3):
        y = y * (_f32(1.5) - _f32(0.5) * a * y * y)
    return y


def _atan01(z):
    u = z * z
    p = _f32(_ATAN_C[8])
    for k in range(7, -1, -1):
        p = p * u + _f32(_ATAN_C[k])
    return p * z


def _qmul(a, b):
    ax, ay, az, aw = a
    bx, by, bz, bw = b
    return (aw * bx + ax * bw + ay * bz - az * by,
            aw * by - ax * bz + ay * bw + az * bx,
            aw * bz + ax * by - ay * bx + az * bw,
            aw * bw - ax * bx - ay * by - az * bz)


def _qrot(q, v):
    qx, qy, qz, qw = q
    vx, vy, vz = v
    ux = qy * vz - qz * vy
    uy = qz * vx - qx * vz
    uz = qx * vy - qy * vx
    wx = qy * uz - qz * uy
    wy = qz * ux - qx * uz
    wz = qx * uy - qy * ux
    return (vx + _f32(2.0) * (qw * ux + wx),
            vy + _f32(2.0) * (qw * uy + wy),
            vz + _f32(2.0) * (qw * uz + wz))


def _edge_math(tp, qp, t1, q1, t2, q2, info):
    qpc = (-qp[0], -qp[1], -qp[2], qp[3])
    q1c = (-q1[0], -q1[1], -q1[2], q1[3])
    u = (t2[0] - t1[0], t2[1] - t1[1], t2[2] - t1[2])
    v1 = _qrot(q1c, u)
    wv = (v1[0] - tp[0], v1[1] - tp[1], v1[2] - tp[2])
    tT = _qrot(qpc, wv)
    qT = _qmul(_qmul(qpc, q1c), q2)

    x, y, z, w = qT
    sgn = jnp.where(w < _f32(0.0), _f32(-1.0), _f32(1.0))
    x = x * sgn
    y = y * sgn
    z = z * sgn
    w = w * sgn
    n2 = x * x + y * y + z * z
    a = n2 + _f32(1e-24)
    ry = _rsqrt(a)
    n = a * ry
    mn = jnp.minimum(n, w)
    mx = jnp.maximum(n, w)
    t = _atan01(mn * _recip(mx))
    half = jnp.where(n > w, _f32(_HALF_PI) - t, t)
    angle = _f32(2.0) * half
    small = n < _f32(1e-7)
    factor = jnp.where(small, _f32(2.0), angle * ry)
    px = x * factor
    py = y * factor
    pz = z * factor
    theta2 = px * px + py * py + pz * pz
    small2 = angle < _f32(1e-6)
    th = jnp.where(small2, _f32(1.0), angle)
    rth = _recip(th)
    coef = jnp.where(small2, _f32(1.0 / 12.0),
                     rth * rth - _f32(0.5) * w * ry * rth)
    tx, ty, tz = tT
    pt = px * tx + py * ty + pz * tz
    aa = _f32(1.0) - coef * theta2
    cx = py * tz - pz * ty
    cy = pz * tx - px * tz
    cz = px * ty - py * tx
    cp = coef * pt
    r0 = tx * aa - _f32(0.5) * cx + cp * px
    r1 = ty * aa - _f32(0.5) * cy + cp * py
    r2 = tz * aa - _f32(0.5) * cz + cp * pz
    r6 = (r0, r1, r2, px, py, pz)
    outs = []
    for i in range(6):
        s = info[i * 6] * r6[0]
        for j in range(1, 6):
            s = s + info[i * 6 + j] * r6[j]
        outs.append(s)
    return outs


def _sc_body(idx1_hbm, idx2_hbm, poses_hbm, infos_hbm, nodes_hbm, out_hbm,
             idx1_v, idx2_v, pos_v, inf_v, n1_v, n2_v, out_v, sem):
    cid = lax.axis_index("c")
    sid = lax.axis_index("s")
    wid = sid * 2 + cid
    iota = lax.iota(jnp.int32, 16)
    i7 = iota * 7
    i36 = iota * 36
    i6 = iota * 6
    cols = [jnp.full((16,), c, jnp.int32) for c in range(7)]

    def chunk_body(i, _):
        c = wid + i * _NW

        @pl.when(c < _NCHUNKS)
        def _():
            base = c * _C
            pltpu.sync_copy(idx1_hbm.at[pl.ds(c * (_C // _S), _C // _S)], idx1_v)
            pltpu.sync_copy(idx2_hbm.at[pl.ds(c * (_C // _S), _C // _S)], idx2_v)
            pltpu.sync_copy(poses_hbm.at[pl.ds(base * 7, _C * 7)], pos_v)
            pltpu.sync_copy(infos_hbm.at[pl.ds(base * 36, _C * 36)], inf_v)
            cps = []
            for k in range(_C // _S):
                cps.append(pltpu.async_copy(
                    nodes_hbm.at[idx1_v.at[k]], n1_v.at[pl.ds(k * _S, _S)], sem))
                cps.append(pltpu.async_copy(
                    nodes_hbm.at[idx2_v.at[k]], n2_v.at[pl.ds(k * _S, _S)], sem))
            for cp in cps:
                cp.wait()

            def group(g, _):
                e0 = g * 16
                rows = iota + e0
                b7 = i7 + e0 * 7
                b36 = i36 + e0 * 36
                b6 = i6 + e0 * 6
                tp = [plsc.load_gather(pos_v, [b7 + c]) for c in range(3)]
                qp = [plsc.load_gather(pos_v, [b7 + c]) for c in range(3, 7)]
                nn1 = [plsc.load_gather(n1_v, [rows, cols[c]]) for c in range(7)]
                nn2 = [plsc.load_gather(n2_v, [rows, cols[c]]) for c in range(7)]
                info = [plsc.load_gather(inf_v, [b36 + c]) for c in range(36)]
                outs = _edge_math(tp, qp, nn1[:3], nn1[3:], nn2[:3], nn2[3:], info)
                for oi in range(6):
                    plsc.store_scatter(out_v, [b6 + oi], outs[oi])

            lax.fori_loop(0, _G, group, None)
            pltpu.sync_copy(out_v, out_hbm.at[pl.ds(base * 6, _C * 6)])

    lax.fori_loop(0, _CPW, chunk_body, None)


_mesh = plsc.VectorSubcoreMesh(core_axis_name="c", subcore_axis_name="s")

_sc_call = functools.partial(
    pl.kernel,
    out_type=jax.ShapeDtypeStruct((_N_EDGES * 6,), jnp.float32),
    mesh=_mesh,
    scratch_types=[
        pltpu.VMEM((_C // _S, _S), jnp.int32),
        pltpu.VMEM((_C // _S, _S), jnp.int32),
        pltpu.VMEM((_C * 7,), jnp.float32),
        pltpu.VMEM((_C * 36,), jnp.float32),
        pltpu.VMEM((_C, 16), jnp.float32),
        pltpu.VMEM((_C, 16), jnp.float32),
        pltpu.VMEM((_C * 6,), jnp.float32),
        pltpu.SemaphoreType.DMA,
    ],
    compiler_params=pltpu.CompilerParams(
        needs_layout_passes=False, use_tc_tiling_on_sc=False),
)(_sc_body)


def kernel(edges, poses, infos, nodes):
    edges = edges.astype(jnp.int32)
    idx1 = edges[:, 0].reshape(_N_EDGES // _S, _S)
    idx2 = edges[:, 1].reshape(_N_EDGES // _S, _S)
    poses_f = poses.reshape(-1)
    infos_f = infos.reshape(-1)
    nodes_p = jnp.pad(nodes, ((0, 0), (0, 9)))
    out = _sc_call(idx1, idx2, poses_f, infos_f, nodes_p)
    return out.reshape(_N_EDGES, 6)


# speedup vs baseline: 2.2059x; 1.0332x over previous
import functools

import jax
import jax.numpy as jnp
from jax import lax
from jax.experimental import pallas as pl
from jax.experimental.pallas import tpu as pltpu
from jax.experimental.pallas import tpu_sc as plsc

_N_NODES = 100000
_N_EDGES = 1600000
_NW = 32
_C = 512
_S = 128
_G = _C // 16
_NCHUNKS = _N_EDGES // _C
_CPW = -(-_NCHUNKS // _NW)

_HALF_PI = 1.5707963267948966

_ATAN_C = (
    0.9999999937572801, -0.3333313797588716, 0.19993694079563193,
    -0.14211102809331508, 0.10667470050577645, -0.07556856634693432,
    0.04327764436731928, -0.01641277490593999, 0.00293264657519318,
)


def _f32(x):
    return jnp.float32(x)


def _recip(x):
    i = plsc.bitcast(x, jnp.int32)
    i = jnp.int32(0x7EF311C3) - i
    r = plsc.bitcast(i, jnp.float32)
    for _ in range(3):
        r = r * (_f32(2.0) - x * r)
    return r


def _rsqrt(a):
    i = plsc.bitcast(a, jnp.int32)
    i = jnp.int32(0x5F3759DF) - (i >> 1)
    y = plsc.bitcast(i, jnp.float32)
    for _ in range(3):
        y = y * (_f32(1.5) - _f32(0.5) * a * y * y)
    return y


def _atan01(z):
    u = z * z
    p = _f32(_ATAN_C[8])
    for k in range(7, -1, -1):
        p = p * u + _f32(_ATAN_C[k])
    return p * z


def _qmul(a, b):
    ax, ay, az, aw = a
    bx, by, bz, bw = b
    return (aw * bx + ax * bw + ay * bz - az * by,
            aw * by - ax * bz + ay * bw + az * bx,
            aw * bz + ax * by - ay * bx + az * bw,
            aw * bw - ax * bx - ay * by - az * bz)


def _qrot(q, v):
    qx, qy, qz, qw = q
    vx, vy, vz = v
    ux = qy * vz - qz * vy
    uy = qz * vx - qx * vz
    uz = qx * vy - qy * vx
    wx = qy * uz - qz * uy
    wy = qz * ux - qx * uz
    wz = qx * uy - qy * ux
    return (vx + _f32(2.0) * (qw * ux + wx),
            vy + _f32(2.0) * (qw * uy + wy),
            vz + _f32(2.0) * (qw * uz + wz))


def _edge_math(tp, qp, t1, q1, t2, q2):
    qpc = (-qp[0], -qp[1], -qp[2], qp[3])
    q1c = (-q1[0], -q1[1], -q1[2], q1[3])
    u = (t2[0] - t1[0], t2[1] - t1[1], t2[2] - t1[2])
    v1 = _qrot(q1c, u)
    wv = (v1[0] - tp[0], v1[1] - tp[1], v1[2] - tp[2])
    tT = _qrot(qpc, wv)
    qT = _qmul(_qmul(qpc, q1c), q2)

    x, y, z, w = qT
    sgn = jnp.where(w < _f32(0.0), _f32(-1.0), _f32(1.0))
    x = x * sgn
    y = y * sgn
    z = z * sgn
    w = w * sgn
    n2 = x * x + y * y + z * z
    a = n2 + _f32(1e-24)
    ry = _rsqrt(a)
    n = a * ry
    mn = jnp.minimum(n, w)
    mx = jnp.maximum(n, w)
    t = _atan01(mn * _recip(mx))
    half = jnp.where(n > w, _f32(_HALF_PI) - t, t)
    angle = _f32(2.0) * half
    small = n < _f32(1e-7)
    factor = jnp.where(small, _f32(2.0), angle * ry)
    px = x * factor
    py = y * factor
    pz = z * factor
    theta2 = px * px + py * py + pz * pz
    small2 = angle < _f32(1e-6)
    th = jnp.where(small2, _f32(1.0), angle)
    rth = _recip(th)
    coef = jnp.where(small2, _f32(1.0 / 12.0),
                     rth * rth - _f32(0.5) * w * ry * rth)
    tx, ty, tz = tT
    pt = px * tx + py * ty + pz * tz
    aa = _f32(1.0) - coef * theta2
    cx = py * tz - pz * ty
    cy = pz * tx - px * tz
    cz = px * ty - py * tx
    cp = coef * pt
    r0 = tx * aa - _f32(0.5) * cx + cp * px
    r1 = ty * aa - _f32(0.5) * cy + cp * py
    r2 = tz * aa - _f32(0.5) * cz + cp * pz
    return (r0, r1, r2, px, py, pz)


def _sc_body(idx1_hbm, idx2_hbm, poses_hbm, infos_hbm, nodes_hbm, out_hbm,
             *scratch):
    set_a = scratch[:8]
    set_b = scratch[8:]
    cid = lax.axis_index("c")
    sid = lax.axis_index("s")
    wid = sid * 2 + cid
    iota = lax.iota(jnp.int32, 16)
    i7 = iota * 7
    i36 = iota * 36
    i6 = iota * 6
    cols = [jnp.full((16,), c, jnp.int32) for c in range(7)]

    def fire(st, c):
        idx1_v, idx2_v, pos_v, inf_v, n1_v, n2_v, _, sem = st
        base = c * _C
        a = pltpu.async_copy(idx1_hbm.at[pl.ds(c * (_C // _S), _C // _S)], idx1_v, sem)
        b = pltpu.async_copy(idx2_hbm.at[pl.ds(c * (_C // _S), _C // _S)], idx2_v, sem)
        a.wait()
        b.wait()
        pltpu.async_copy(poses_hbm.at[pl.ds(base * 7, _C * 7)], pos_v, sem)
        pltpu.async_copy(infos_hbm.at[pl.ds(base * 36, _C * 36)], inf_v, sem)
        for k in range(_C // _S):
            pltpu.async_copy(nodes_hbm.at[idx1_v.at[k]], n1_v.at[pl.ds(k * _S, _S)], sem)
            pltpu.async_copy(nodes_hbm.at[idx2_v.at[k]], n2_v.at[pl.ds(k * _S, _S)], sem)

    def drain(st, c):
        idx1_v, idx2_v, pos_v, inf_v, n1_v, n2_v, _, sem = st
        base = c * _C
        pltpu.make_async_copy(poses_hbm.at[pl.ds(base * 7, _C * 7)], pos_v, sem).wait()
        pltpu.make_async_copy(infos_hbm.at[pl.ds(base * 36, _C * 36)], inf_v, sem).wait()
        for k in range(_C // _S):
            pltpu.make_async_copy(
                nodes_hbm.at[idx1_v.at[k]], n1_v.at[pl.ds(k * _S, _S)], sem).wait()
            pltpu.make_async_copy(
                nodes_hbm.at[idx2_v.at[k]], n2_v.at[pl.ds(k * _S, _S)], sem).wait()

    def compute(st, c):
        _, _, pos_v, inf_v, n1_v, n2_v, out_v, _ = st
        base = c * _C

        def group(g, _):
            e0 = g * 16
            rows = iota + e0
            b7 = i7 + e0 * 7
            b36 = i36 + e0 * 36
            b6 = i6 + e0 * 6
            tp = [plsc.load_gather(pos_v, [b7 + c2]) for c2 in range(3)]
            qp = [plsc.load_gather(pos_v, [b7 + c2]) for c2 in range(3, 7)]
            nn1 = [plsc.load_gather(n1_v, [rows, cols[c2]]) for c2 in range(7)]
            nn2 = [plsc.load_gather(n2_v, [rows, cols[c2]]) for c2 in range(7)]
            r6 = _edge_math(tp, qp, nn1[:3], nn1[3:], nn2[:3], nn2[3:])
            for oi in range(6):
                s = plsc.load_gather(inf_v, [b36 + oi * 6]) * r6[0]
                for j in range(1, 6):
                    s = s + plsc.load_gather(inf_v, [b36 + (oi * 6 + j)]) * r6[j]
                plsc.store_scatter(out_v, [b6 + oi], s)

        lax.fori_loop(0, _G, group, None)
        pltpu.sync_copy(out_v, out_hbm.at[pl.ds(base * 6, _C * 6)])

    fire(set_a, wid)

    def pair_body(j, _):
        c0 = wid + (2 * j) * _NW
        c1 = c0 + _NW
        c2 = c0 + 2 * _NW

        @pl.when(c1 < _NCHUNKS)
        def _():
            fire(set_b, c1)

        drain(set_a, c0)
        compute(set_a, c0)

        @pl.when(c2 < _NCHUNKS)
        def _():
            fire(set_a, c2)

        @pl.when(c1 < _NCHUNKS)
        def _():
            drain(set_b, c1)
            compute(set_b, c1)

    lax.fori_loop(0, _CPW // 2, pair_body, None)


_mesh = plsc.VectorSubcoreMesh(core_axis_name="c", subcore_axis_name="s")

_sc_call = functools.partial(
    pl.kernel,
    out_type=jax.ShapeDtypeStruct((_N_EDGES * 6,), jnp.float32),
    mesh=_mesh,
    scratch_types=[
        pltpu.VMEM((_C // _S, _S), jnp.int32),
        pltpu.VMEM((_C // _S, _S), jnp.int32),
        pltpu.VMEM((_C * 7,), jnp.float32),
        pltpu.VMEM((_C * 36,), jnp.float32),
        pltpu.VMEM((_C, 16), jnp.float32),
        pltpu.VMEM((_C, 16), jnp.float32),
        pltpu.VMEM((_C * 6,), jnp.float32),
        pltpu.SemaphoreType.DMA,
    ] * 2,
    compiler_params=pltpu.CompilerParams(
        needs_layout_passes=False, use_tc_tiling_on_sc=False),
)(_sc_body)


def kernel(edges, poses, infos, nodes):
    edges = edges.astype(jnp.int32)
    idx1 = edges[:, 0].reshape(_N_EDGES // _S, _S)
    idx2 = edges[:, 1].reshape(_N_EDGES // _S, _S)
    poses_f = poses.reshape(-1)
    infos_f = infos.reshape(-1)
    nodes_p = jnp.pad(nodes, ((0, 0), (0, 9)))
    out = _sc_call(idx1, idx2, poses_f, infos_f, nodes_p)
    return out.reshape(_N_EDGES, 6)
